# Initial kernel scaffold; baseline (speedup 1.0000x reference)
import jax
import jax.numpy as jnp
from jax.experimental import pallas as pl


def kernel(x, edge_index, W1, a_src, a_dst, b1, W2, b2):
    raise NotImplementedError("write your pallas kernel here")


# speedup vs baseline: 16.7414x; 16.7414x over previous
import functools

import jax
import jax.numpy as jnp
from jax import lax
from jax.experimental import pallas as pl
from jax.experimental.pallas import tpu as pltpu
from jax.experimental.pallas import tpu_sc as plsc

N = 10000
E = 320000
E2 = E + N
D_IN = 128
H = 256
D_OUT = 128

NC = 2
NS = 16
BLK = 128
BLK1 = 64
EPS = 20736
NBLK = EPS // BLK1
EPS2 = 10496
NBLK2 = EPS2 // BLK
E2PAD = EPS2 * 32
NTAB = 10016
NACC = 10016
DACC = 10240

_F32 = jnp.float32
_I32 = jnp.int32


def _bcast_lane(vec16, j):
    idx = jnp.full((16, 1), j, _I32)
    return lax.gather(
        vec16, idx,
        lax.GatherDimensionNumbers(offset_dims=(), collapsed_slice_dims=(0,),
                                   start_index_map=(0,)),
        slice_sizes=(1,), mode=lax.GatherScatterMode.PROMISE_IN_BOUNDS)


def _mm(a, b):
    return lax.dot_general(a, b, (((1,), (0,)), ((), ())),
                           precision=lax.Precision.HIGHEST,
                           preferred_element_type=_F32)


def _tc1_body(x_ref, w1_ref, a2_ref, hcat_ref, ea_ref):
    j = pl.program_id(1)
    h = _mm(x_ref[...], w1_ref[...])
    hcat_ref[...] = h
    part = _mm(h, a2_ref[...])

    @pl.when(j == 0)
    def _():
        ea_ref[...] = part

    @pl.when(j != 0)
    def _():
        ea_ref[...] = ea_ref[...] + part


def _tc1(x, W1, A2):
    mb = 1000
    return pl.pallas_call(
        _tc1_body,
        grid=(N // mb, 2),
        in_specs=[
            pl.BlockSpec((mb, D_IN), lambda i, j: (i, 0)),
            pl.BlockSpec((D_IN, H // 2), lambda i, j: (0, j)),
            pl.BlockSpec((H // 2, 2), lambda i, j: (j, 0)),
        ],
        out_specs=[
            pl.BlockSpec((mb, D_IN), lambda i, j: (j * (N // mb) + i, 0)),
            pl.BlockSpec((mb, 2), lambda i, j: (i, 0)),
        ],
        out_shape=[
            jax.ShapeDtypeStruct((2 * N, D_IN), _F32),
            jax.ShapeDtypeStruct((N, 2), _F32),
        ],
        compiler_params=pltpu.CompilerParams(
            dimension_semantics=("parallel", "arbitrary")),
    )(x, W1, A2)


def _sc1_body(src_hbm, dst_hbm, es_hbm, ed_hbm, hcat_hbm,
              numerA, numerB, den_out, deg_out,
              es_t, ed_t,
              sr0, sr1, ssc0, ssc1, sa0, sa1,
              dr0, dr1, dsc0, dsc1, pv0, pv1, ones_v, zrow,
              rows0, rows1, acc, dacc, gacc,
              gsem0, gsem1, isem0, isem1):
    c = lax.axis_index("c")
    s = lax.axis_index("s")

    pltpu.sync_copy(es_hbm, es_t)
    pltpu.sync_copy(ed_hbm, ed_t)

    zero16 = jnp.zeros((16,), _F32)
    one16 = jnp.ones((16,), _F32)
    for v in range(8):
        zrow[0, pl.ds(v * 16, 16)] = zero16
    for v in range(BLK1 // 16):
        ones_v[pl.ds(v * 16, 16)] = one16

    def _zrows(r, carry):
        for v in range(8):
            rows0[r, pl.ds(v * 16, 16)] = zero16
        return carry
    lax.fori_loop(0, BLK1, _zrows, 0)

    r0 = pl.multiple_of(s * 632, 8)

    @pl.when(s < 15)
    def _():
        for k in range(9):
            pltpu.sync_copy(rows0, acc.at[pl.ds(r0 + k * 64, 64)])
        pltpu.sync_copy(rows0.at[pl.ds(0, 56)], acc.at[pl.ds(r0 + 576, 56)])

    @pl.when(s == 15)
    def _():
        for k in range(8):
            pltpu.sync_copy(rows0, acc.at[pl.ds(9480 + k * 64, 64)])
        pltpu.sync_copy(rows0.at[pl.ds(0, 24)], acc.at[pl.ds(9992, 24)])

    for k in range(5):
        off = pl.multiple_of(s * 640 + k * 128, 128)
        pltpu.sync_copy(zrow.at[0], dacc.at[pl.ds(off, 128)])
        pltpu.sync_copy(zrow.at[0], gacc.at[pl.ds(off, 128)])
    plsc.subcore_barrier()

    coff = c * N

    def ibase(b):
        return pl.multiple_of(s * EPS + b * BLK1, 64)

    def idx_start(b, sr, dr, isem):
        pltpu.async_copy(src_hbm.at[pl.ds(ibase(b), BLK1)], sr, isem)
        pltpu.async_copy(dst_hbm.at[pl.ds(ibase(b), BLK1)], dr.at[0], isem)

    def idx_wait(b, sr, dr, isem):
        pltpu.make_async_copy(src_hbm.at[pl.ds(ibase(b), BLK1)], sr, isem).wait()
        pltpu.make_async_copy(dst_hbm.at[pl.ds(ibase(b), BLK1)], dr.at[0],
                              isem).wait()

    def stage_idx(sr, dr, ssc, dsc, sa):
        for v in range(BLK1 // 16):
            sv = sr[pl.ds(v * 16, 16)]
            ssc[pl.ds(v * 16, 16)] = sv
            sa[pl.ds(v * 16, 16)] = sv + coff
            dsc[0, pl.ds(v * 16, 16)] = dr[0, pl.ds(v * 16, 16)]

    def gather_start(sa, rows, gsem):
        pltpu.async_copy(hcat_hbm.at[sa], rows, gsem)

    def gather_wait(sa, rows, gsem):
        pltpu.make_async_copy(hcat_hbm.at[sa], rows, gsem).wait()

    def scatter_now(rows, dsc):
        pltpu.sync_copy(rows, acc.at[dsc.at[0]], add=True)

    def compute_scale(ssc, dsc, pvr, rows):
        def inner(v, carry):
            sv = ssc[pl.ds(v * 16, 16)]
            dv = dsc[0, pl.ds(v * 16, 16)]
            t = plsc.load_gather(es_t, [sv]) + plsc.load_gather(ed_t, [dv])
            p16 = jnp.exp(jnp.maximum(t, 0.2 * t))
            pvr[pl.ds(v * 16, 16)] = p16
            for j in range(16):
                pvb = _bcast_lane(p16, j)
                r = v * 16 + j
                for q in range(8):
                    rows[r, pl.ds(q * 16, 16)] = (
                        rows[r, pl.ds(q * 16, 16)] * pvb)
            return carry
        lax.fori_loop(0, BLK1 // 16, inner, 0)

    def small_scatters(pvr, dsc):
        @pl.when(c == 0)
        def _():
            pltpu.sync_copy(pvr, dacc.at[dsc.at[0]], add=True)
            pltpu.sync_copy(ones_v, gacc.at[dsc.at[0]], add=True)

    pltpu.sync_copy(src_hbm.at[pl.ds(ibase(0), BLK1)], sr0)
    pltpu.sync_copy(dst_hbm.at[pl.ds(ibase(0), BLK1)], dr0.at[0])
    stage_idx(sr0, dr0, ssc0, dsc0, sa0)
    gather_start(sa0, rows0, gsem0)
    idx_start(1, sr1, dr1, isem1)

    def _pair(qq, carry):
        b0 = qq * 2
        gather_wait(sa0, rows0, gsem0)
        idx_wait(b0 + 1, sr1, dr1, isem1)
        stage_idx(sr1, dr1, ssc1, dsc1, sa1)
        gather_start(sa1, rows1, gsem1)

        @pl.when(b0 + 2 < NBLK)
        def _():
            idx_start(b0 + 2, sr0, dr0, isem0)
        compute_scale(ssc0, dsc0, pv0, rows0)
        scatter_now(rows0, dsc0)
        small_scatters(pv0, dsc0)

        gather_wait(sa1, rows1, gsem1)

        @pl.when(b0 + 2 < NBLK)
        def _():
            idx_wait(b0 + 2, sr0, dr0, isem0)
            stage_idx(sr0, dr0, ssc0, dsc0, sa0)
            gather_start(sa0, rows0, gsem0)

        @pl.when(b0 + 3 < NBLK)
        def _():
            idx_start(b0 + 3, sr1, dr1, isem1)
        compute_scale(ssc1, dsc1, pv1, rows1)
        scatter_now(rows1, dsc1)
        small_scatters(pv1, dsc1)
        return carry

    lax.fori_loop(0, NBLK // 2, _pair, 0)
    plsc.subcore_barrier()

    q0 = pl.multiple_of(s * 624, 8)

    def _copy_num(dst_ref):
        @pl.when(s < 15)
        def _():
            pltpu.sync_copy(acc.at[pl.ds(q0, 624)], dst_ref.at[pl.ds(q0, 624)])

        @pl.when(s == 15)
        def _():
            pltpu.sync_copy(acc.at[pl.ds(9360, 640)], dst_ref.at[pl.ds(9360, 640)])

    @pl.when(c == 0)
    def _():
        _copy_num(numerA)
        doff = pl.multiple_of(s * 640, 128)
        pltpu.sync_copy(dacc.at[pl.ds(doff, 640)], den_out.at[pl.ds(doff, 640)])
        pltpu.sync_copy(gacc.at[pl.ds(doff, 640)], deg_out.at[pl.ds(doff, 640)])

    @pl.when(c == 1)
    def _():
        _copy_num(numerB)


@functools.partial(
    pl.kernel,
    out_type=[
        jax.ShapeDtypeStruct((N, D_IN), _F32),
        jax.ShapeDtypeStruct((N, D_IN), _F32),
        jax.ShapeDtypeStruct((DACC,), _F32),
        jax.ShapeDtypeStruct((DACC,), _F32),
    ],
    mesh=plsc.VectorSubcoreMesh(core_axis_name="c", subcore_axis_name="s"),
    compiler_params=pltpu.CompilerParams(needs_layout_passes=False),
    scratch_types=[
        pltpu.VMEM((NTAB,), _F32),
        pltpu.VMEM((NTAB,), _F32),
        pltpu.VMEM((BLK1,), _I32),
        pltpu.VMEM((BLK1,), _I32),
        pltpu.VMEM((BLK1,), _I32),
        pltpu.VMEM((BLK1,), _I32),
        pltpu.VMEM((BLK1,), _I32),
        pltpu.VMEM((BLK1,), _I32),
        pltpu.VMEM((1, BLK1), _I32),
        pltpu.VMEM((1, BLK1), _I32),
        pltpu.VMEM((1, BLK1), _I32),
        pltpu.VMEM((1, BLK1), _I32),
        pltpu.VMEM((BLK1,), _F32),
        pltpu.VMEM((BLK1,), _F32),
        pltpu.VMEM((BLK1,), _F32),
        pltpu.VMEM((1, 128), _F32),
        pltpu.VMEM((BLK1, D_IN), _F32),
        pltpu.VMEM((BLK1, D_IN), _F32),
        pltpu.VMEM_SHARED((NACC, D_IN), _F32),
        pltpu.VMEM_SHARED((DACC,), _F32),
        pltpu.VMEM_SHARED((DACC,), _F32),
        pltpu.SemaphoreType.DMA,
        pltpu.SemaphoreType.DMA,
        pltpu.SemaphoreType.DMA,
        pltpu.SemaphoreType.DMA,
    ],
)
def _sc1(*args):
    _sc1_body(*args)


def _tc2_body(nA_ref, nB_ref, den_ref, deg_ref, b1_ref, w2_ref,
              g_ref, dinv_ref):
    x1 = jnp.concatenate([nA_ref[...], nB_ref[...]], axis=1)
    x1 = jnp.maximum(x1 / (den_ref[...] + 1e-16) + b1_ref[...], 0.0)
    dg = deg_ref[...]
    dv = jnp.where(dg > 0, lax.rsqrt(jnp.maximum(dg, 1e-12)), 0.0)
    g_ref[...] = _mm(x1, w2_ref[...]) * dv
    dinv_ref[...] = dv


def _tc2(nA, nB, den, deg, b1r, W2):
    mb = 1000
    return pl.pallas_call(
        _tc2_body,
        grid=(N // mb,),
        in_specs=[
            pl.BlockSpec((mb, D_IN), lambda i: (i, 0)),
            pl.BlockSpec((mb, D_IN), lambda i: (i, 0)),
            pl.BlockSpec((mb, 1), lambda i: (i, 0)),
            pl.BlockSpec((mb, 1), lambda i: (i, 0)),
            pl.BlockSpec((1, H), lambda i: (0, 0)),
            pl.BlockSpec((H, D_OUT), lambda i: (0, 0)),
        ],
        out_specs=[
            pl.BlockSpec((mb, D_OUT), lambda i: (i, 0)),
            pl.BlockSpec((mb, 1), lambda i: (i, 0)),
        ],
        out_shape=[
            jax.ShapeDtypeStruct((N, D_OUT), _F32),
            jax.ShapeDtypeStruct((N, 1), _F32),
        ],
    )(nA, nB, den, deg, b1r, W2)


def _sc2_body(src_hbm, dst_hbm, g_hbm, accA_out, accB_out,
              sr0, sr1, dr0, dr1, dsc0, dsc1, rows0, rows1, acc,
              gsem0, gsem1, isem0, isem1):
    c = lax.axis_index("c")
    s = lax.axis_index("s")

    zero16 = jnp.zeros((16,), _F32)

    def _zrows(r, carry):
        for v in range(8):
            rows0[r, pl.ds(v * 16, 16)] = zero16
        return carry
    lax.fori_loop(0, BLK, _zrows, 0)

    r0 = pl.multiple_of(s * 632, 8)

    @pl.when(s < 15)
    def _():
        for k in range(4):
            pltpu.sync_copy(rows0, acc.at[pl.ds(r0 + k * 128, 128)])
        pltpu.sync_copy(rows0.at[pl.ds(0, 120)], acc.at[pl.ds(r0 + 512, 120)])

    @pl.when(s == 15)
    def _():
        for k in range(4):
            pltpu.sync_copy(rows0, acc.at[pl.ds(9480 + k * 128, 128)])
        pltpu.sync_copy(rows0.at[pl.ds(0, 24)], acc.at[pl.ds(9992, 24)])
    plsc.subcore_barrier()

    w0 = (c * NS + s) * EPS2

    def ibase(b):
        return pl.multiple_of(w0 + b * BLK, 128)

    def idx_start(b, sr, dr, isem):
        pltpu.async_copy(src_hbm.at[pl.ds(ibase(b), BLK)], sr, isem)
        pltpu.async_copy(dst_hbm.at[pl.ds(ibase(b), BLK)], dr.at[0], isem)

    def idx_wait(b, sr, dr, isem):
        pltpu.make_async_copy(src_hbm.at[pl.ds(ibase(b), BLK)], sr, isem).wait()
        pltpu.make_async_copy(dst_hbm.at[pl.ds(ibase(b), BLK)], dr.at[0],
                              isem).wait()

    def stage_dst(dr, dsc):
        for v in range(8):
            dsc[0, pl.ds(v * 16, 16)] = dr[0, pl.ds(v * 16, 16)]

    pltpu.sync_copy(src_hbm.at[pl.ds(ibase(0), BLK)], sr0)
    pltpu.sync_copy(dst_hbm.at[pl.ds(ibase(0), BLK)], dr0.at[0])
    stage_dst(dr0, dsc0)
    pltpu.async_copy(g_hbm.at[sr0], rows0, gsem0)
    idx_start(1, sr1, dr1, isem1)

    def _pair(qq, carry):
        b0 = qq * 2
        pltpu.make_async_copy(g_hbm.at[sr0], rows0, gsem0).wait()
        idx_wait(b0 + 1, sr1, dr1, isem1)
        stage_dst(dr1, dsc1)
        pltpu.async_copy(g_hbm.at[sr1], rows1, gsem1)

        @pl.when(b0 + 2 < NBLK2)
        def _():
            idx_start(b0 + 2, sr0, dr0, isem0)
        pltpu.sync_copy(rows0, acc.at[dsc0.at[0]], add=True)

        pltpu.make_async_copy(g_hbm.at[sr1], rows1, gsem1).wait()

        @pl.when(b0 + 2 < NBLK2)
        def _():
            idx_wait(b0 + 2, sr0, dr0, isem0)
            stage_dst(dr0, dsc0)
            pltpu.async_copy(g_hbm.at[sr0], rows0, gsem0)

        @pl.when(b0 + 3 < NBLK2)
        def _():
            idx_start(b0 + 3, sr1, dr1, isem1)
        pltpu.sync_copy(rows1, acc.at[dsc1.at[0]], add=True)
        return carry

    lax.fori_loop(0, NBLK2 // 2, _pair, 0)
    plsc.subcore_barrier()

    q0 = pl.multiple_of(s * 624, 8)

    def _copy_out(dst_ref):
        @pl.when(s < 15)
        def _():
            pltpu.sync_copy(acc.at[pl.ds(q0, 624)], dst_ref.at[pl.ds(q0, 624)])

        @pl.when(s == 15)
        def _():
            pltpu.sync_copy(acc.at[pl.ds(9360, 640)], dst_ref.at[pl.ds(9360, 640)])

    @pl.when(c == 0)
    def _():
        _copy_out(accA_out)

    @pl.when(c == 1)
    def _():
        _copy_out(accB_out)


@functools.partial(
    pl.kernel,
    out_type=[
        jax.ShapeDtypeStruct((N, D_OUT), _F32),
        jax.ShapeDtypeStruct((N, D_OUT), _F32),
    ],
    mesh=plsc.VectorSubcoreMesh(core_axis_name="c", subcore_axis_name="s"),
    compiler_params=pltpu.CompilerParams(needs_layout_passes=False),
    scratch_types=[
        pltpu.VMEM((BLK,), _I32),
        pltpu.VMEM((BLK,), _I32),
        pltpu.VMEM((1, BLK), _I32),
        pltpu.VMEM((1, BLK), _I32),
        pltpu.VMEM((1, BLK), _I32),
        pltpu.VMEM((1, BLK), _I32),
        pltpu.VMEM((BLK, D_OUT), _F32),
        pltpu.VMEM((BLK, D_OUT), _F32),
        pltpu.VMEM_SHARED((NACC, D_OUT), _F32),
        pltpu.SemaphoreType.DMA,
        pltpu.SemaphoreType.DMA,
        pltpu.SemaphoreType.DMA,
        pltpu.SemaphoreType.DMA,
    ],
)
def _sc2(*args):
    _sc2_body(*args)


def _tc3_body(aA_ref, aB_ref, dinv_ref, b2_ref, z_ref):
    z_ref[...] = ((aA_ref[...] + aB_ref[...]) * dinv_ref[...] + b2_ref[...])


def _tc3(aA, aB, dinv, b2r):
    mb = 1000
    return pl.pallas_call(
        _tc3_body,
        grid=(N // mb,),
        in_specs=[
            pl.BlockSpec((mb, D_OUT), lambda i: (i, 0)),
            pl.BlockSpec((mb, D_OUT), lambda i: (i, 0)),
            pl.BlockSpec((mb, 1), lambda i: (i, 0)),
            pl.BlockSpec((1, D_OUT), lambda i: (0, 0)),
        ],
        out_specs=pl.BlockSpec((mb, D_OUT), lambda i: (i, 0)),
        out_shape=jax.ShapeDtypeStruct((N, D_OUT), _F32),
    )(aA, aB, dinv, b2r)


def kernel(x, edge_index, W1, a_src, a_dst, b1, W2, b2):
    loop = jnp.arange(N, dtype=_I32)
    src = jnp.concatenate([edge_index[0].astype(_I32), loop])
    dst = jnp.concatenate([edge_index[1].astype(_I32), loop])
    npad = E2PAD - E2
    src_pad = jnp.concatenate([src, jnp.zeros((npad,), _I32)])
    dst_pad = jnp.concatenate([dst, jnp.full((npad,), N, _I32)])

    A2 = jnp.stack([a_src, a_dst], axis=1)
    hcat, ea = _tc1(x, W1, A2)
    es_pad = jnp.zeros((NTAB,), _F32).at[:N].set(ea[:, 0])
    ed_pad = jnp.zeros((NTAB,), _F32).at[:N].set(ea[:, 1])

    numA, numB, den2, deg2 = _sc1(src_pad, dst_pad, es_pad, ed_pad, hcat)
    den = den2[:N].reshape(N, 1)
    deg = deg2[:N].reshape(N, 1)

    g, dinv = _tc2(numA, numB, den, deg, b1.reshape(1, H), W2)
    accA, accB = _sc2(src_pad, dst_pad, g)
    return _tc3(accA, accB, dinv, b2.reshape(1, D_OUT))


# speedup vs baseline: 16.7502x; 1.0005x over previous
import functools

import jax
import jax.numpy as jnp
from jax import lax
from jax.experimental import pallas as pl
from jax.experimental.pallas import tpu as pltpu
from jax.experimental.pallas import tpu_sc as plsc

N = 10000
E = 320000
E2 = E + N
D_IN = 128
H = 256
D_OUT = 128

NC = 2
NS = 16
BLK = 128
BLK1 = 64
EPS = 20736
NBLK = EPS // BLK1
EPS2 = 10496
NBLK2 = EPS2 // BLK
E2PAD = EPS2 * 32
NTAB = 10016
NACC = 10016
DACC = 10240

_F32 = jnp.float32
_I32 = jnp.int32


def _bcast_lane(vec16, j):
    idx = jnp.full((16, 1), j, _I32)
    return lax.gather(
        vec16, idx,
        lax.GatherDimensionNumbers(offset_dims=(), collapsed_slice_dims=(0,),
                                   start_index_map=(0,)),
        slice_sizes=(1,), mode=lax.GatherScatterMode.PROMISE_IN_BOUNDS)


def _mm(a, b):
    return lax.dot_general(a, b, (((1,), (0,)), ((), ())),
                           precision=lax.Precision.HIGHEST,
                           preferred_element_type=_F32)


def _tc1_body(x_ref, w1_ref, a2_ref, hcat_ref, ea_ref):
    j = pl.program_id(1)
    h = _mm(x_ref[...], w1_ref[...])
    hcat_ref[...] = h
    part = _mm(h, a2_ref[...])

    @pl.when(j == 0)
    def _():
        ea_ref[...] = part

    @pl.when(j != 0)
    def _():
        ea_ref[...] = ea_ref[...] + part


def _tc1(x, W1, A2):
    mb = 1000
    return pl.pallas_call(
        _tc1_body,
        grid=(N // mb, 2),
        in_specs=[
            pl.BlockSpec((mb, D_IN), lambda i, j: (i, 0)),
            pl.BlockSpec((D_IN, H // 2), lambda i, j: (0, j)),
            pl.BlockSpec((H // 2, 2), lambda i, j: (j, 0)),
        ],
        out_specs=[
            pl.BlockSpec((mb, D_IN), lambda i, j: (j * (N // mb) + i, 0)),
            pl.BlockSpec((mb, 2), lambda i, j: (i, 0)),
        ],
        out_shape=[
            jax.ShapeDtypeStruct((2 * N, D_IN), _F32),
            jax.ShapeDtypeStruct((N, 2), _F32),
        ],
        compiler_params=pltpu.CompilerParams(
            dimension_semantics=("parallel", "arbitrary")),
    )(x, W1, A2)


def _sc1_body(src_hbm, dst_hbm, es_hbm, ed_hbm, hcat_hbm,
              numerA, numerB, den_out, deg_out,
              es_t, ed_t,
              sr0, sr1, ssc0, ssc1, sa0, sa1,
              dr0, dr1, dsc0, dsc1, pv0, pv1, ones_v, zrow,
              rows0, rows1, acc, dacc, gacc,
              gsem0, gsem1, ssem0, ssem1, isem0, isem1):
    c = lax.axis_index("c")
    s = lax.axis_index("s")

    pltpu.sync_copy(es_hbm, es_t)
    pltpu.sync_copy(ed_hbm, ed_t)

    zero16 = jnp.zeros((16,), _F32)
    one16 = jnp.ones((16,), _F32)
    for v in range(8):
        zrow[0, pl.ds(v * 16, 16)] = zero16
    for v in range(BLK1 // 16):
        ones_v[pl.ds(v * 16, 16)] = one16

    def _zrows(r, carry):
        for v in range(8):
            rows0[r, pl.ds(v * 16, 16)] = zero16
        return carry
    lax.fori_loop(0, BLK1, _zrows, 0)

    r0 = pl.multiple_of(s * 632, 8)

    @pl.when(s < 15)
    def _():
        for k in range(9):
            pltpu.sync_copy(rows0, acc.at[pl.ds(r0 + k * 64, 64)])
        pltpu.sync_copy(rows0.at[pl.ds(0, 56)], acc.at[pl.ds(r0 + 576, 56)])

    @pl.when(s == 15)
    def _():
        for k in range(8):
            pltpu.sync_copy(rows0, acc.at[pl.ds(9480 + k * 64, 64)])
        pltpu.sync_copy(rows0.at[pl.ds(0, 24)], acc.at[pl.ds(9992, 24)])

    for k in range(5):
        off = pl.multiple_of(s * 640 + k * 128, 128)
        pltpu.sync_copy(zrow.at[0], dacc.at[pl.ds(off, 128)])
        pltpu.sync_copy(zrow.at[0], gacc.at[pl.ds(off, 128)])
    plsc.subcore_barrier()

    coff = c * N

    def ibase(b):
        return pl.multiple_of(s * EPS + b * BLK1, 64)

    def idx_start(b, sr, dr, isem):
        pltpu.async_copy(src_hbm.at[pl.ds(ibase(b), BLK1)], sr, isem)
        pltpu.async_copy(dst_hbm.at[pl.ds(ibase(b), BLK1)], dr.at[0], isem)

    def idx_wait(b, sr, dr, isem):
        pltpu.make_async_copy(src_hbm.at[pl.ds(ibase(b), BLK1)], sr, isem).wait()
        pltpu.make_async_copy(dst_hbm.at[pl.ds(ibase(b), BLK1)], dr.at[0],
                              isem).wait()

    def stage_idx(sr, dr, ssc, dsc, sa):
        for v in range(BLK1 // 16):
            sv = sr[pl.ds(v * 16, 16)]
            ssc[pl.ds(v * 16, 16)] = sv
            sa[pl.ds(v * 16, 16)] = sv + coff
            dsc[0, pl.ds(v * 16, 16)] = dr[0, pl.ds(v * 16, 16)]

    def gather_start(sa, rows, gsem):
        pltpu.async_copy(hcat_hbm.at[sa], rows, gsem)

    def gather_wait(sa, rows, gsem):
        pltpu.make_async_copy(hcat_hbm.at[sa], rows, gsem).wait()

    def scatter_start(rows, dsc, ssem):
        pltpu.async_copy(rows, acc.at[dsc.at[0]], ssem, add=True)

    def scatter_wait(rows, dsc, ssem):
        pltpu.make_async_copy(rows, acc.at[dsc.at[0]], ssem).wait()

    def compute_scale(ssc, dsc, pvr, rows):
        def inner(v, carry):
            sv = ssc[pl.ds(v * 16, 16)]
            dv = dsc[0, pl.ds(v * 16, 16)]
            t = plsc.load_gather(es_t, [sv]) + plsc.load_gather(ed_t, [dv])
            p16 = jnp.exp(jnp.maximum(t, 0.2 * t))
            pvr[pl.ds(v * 16, 16)] = p16
            for j in range(16):
                pvb = _bcast_lane(p16, j)
                r = v * 16 + j
                for q in range(8):
                    rows[r, pl.ds(q * 16, 16)] = (
                        rows[r, pl.ds(q * 16, 16)] * pvb)
            return carry
        lax.fori_loop(0, BLK1 // 16, inner, 0)

    def small_scatters(pvr, dsc):
        @pl.when(c == 0)
        def _():
            pltpu.sync_copy(pvr, dacc.at[dsc.at[0]], add=True)
            pltpu.sync_copy(ones_v, gacc.at[dsc.at[0]], add=True)

    pltpu.sync_copy(src_hbm.at[pl.ds(ibase(0), BLK1)], sr0)
    pltpu.sync_copy(dst_hbm.at[pl.ds(ibase(0), BLK1)], dr0.at[0])
    stage_idx(sr0, dr0, ssc0, dsc0, sa0)
    gather_start(sa0, rows0, gsem0)
    idx_start(1, sr1, dr1, isem1)

    def _pair(qq, carry):
        b0 = qq * 2
        gather_wait(sa0, rows0, gsem0)

        @pl.when(b0 > 0)
        def _():
            scatter_wait(rows1, dsc1, ssem1)
        idx_wait(b0 + 1, sr1, dr1, isem1)
        stage_idx(sr1, dr1, ssc1, dsc1, sa1)
        gather_start(sa1, rows1, gsem1)

        @pl.when(b0 + 2 < NBLK)
        def _():
            idx_start(b0 + 2, sr0, dr0, isem0)
        compute_scale(ssc0, dsc0, pv0, rows0)
        scatter_start(rows0, dsc0, ssem0)
        small_scatters(pv0, dsc0)

        gather_wait(sa1, rows1, gsem1)
        scatter_wait(rows0, dsc0, ssem0)

        @pl.when(b0 + 2 < NBLK)
        def _():
            idx_wait(b0 + 2, sr0, dr0, isem0)
            stage_idx(sr0, dr0, ssc0, dsc0, sa0)
            gather_start(sa0, rows0, gsem0)

        @pl.when(b0 + 3 < NBLK)
        def _():
            idx_start(b0 + 3, sr1, dr1, isem1)
        compute_scale(ssc1, dsc1, pv1, rows1)
        scatter_start(rows1, dsc1, ssem1)
        small_scatters(pv1, dsc1)
        return carry

    lax.fori_loop(0, NBLK // 2, _pair, 0)
    scatter_wait(rows1, dsc1, ssem1)
    plsc.subcore_barrier()

    q0 = pl.multiple_of(s * 624, 8)

    def _copy_num(dst_ref):
        @pl.when(s < 15)
        def _():
            pltpu.sync_copy(acc.at[pl.ds(q0, 624)], dst_ref.at[pl.ds(q0, 624)])

        @pl.when(s == 15)
        def _():
            pltpu.sync_copy(acc.at[pl.ds(9360, 640)], dst_ref.at[pl.ds(9360, 640)])

    @pl.when(c == 0)
    def _():
        _copy_num(numerA)
        doff = pl.multiple_of(s * 640, 128)
        pltpu.sync_copy(dacc.at[pl.ds(doff, 640)], den_out.at[pl.ds(doff, 640)])
        pltpu.sync_copy(gacc.at[pl.ds(doff, 640)], deg_out.at[pl.ds(doff, 640)])

    @pl.when(c == 1)
    def _():
        _copy_num(numerB)


@functools.partial(
    pl.kernel,
    out_type=[
        jax.ShapeDtypeStruct((N, D_IN), _F32),
        jax.ShapeDtypeStruct((N, D_IN), _F32),
        jax.ShapeDtypeStruct((DACC,), _F32),
        jax.ShapeDtypeStruct((DACC,), _F32),
    ],
    mesh=plsc.VectorSubcoreMesh(core_axis_name="c", subcore_axis_name="s"),
    compiler_params=pltpu.CompilerParams(needs_layout_passes=False),
    scratch_types=[
        pltpu.VMEM((NTAB,), _F32),
        pltpu.VMEM((NTAB,), _F32),
        pltpu.VMEM((BLK1,), _I32),
        pltpu.VMEM((BLK1,), _I32),
        pltpu.VMEM((BLK1,), _I32),
        pltpu.VMEM((BLK1,), _I32),
        pltpu.VMEM((BLK1,), _I32),
        pltpu.VMEM((BLK1,), _I32),
        pltpu.VMEM((1, BLK1), _I32),
        pltpu.VMEM((1, BLK1), _I32),
        pltpu.VMEM((1, BLK1), _I32),
        pltpu.VMEM((1, BLK1), _I32),
        pltpu.VMEM((BLK1,), _F32),
        pltpu.VMEM((BLK1,), _F32),
        pltpu.VMEM((BLK1,), _F32),
        pltpu.VMEM((1, 128), _F32),
        pltpu.VMEM((BLK1, D_IN), _F32),
        pltpu.VMEM((BLK1, D_IN), _F32),
        pltpu.VMEM_SHARED((NACC, D_IN), _F32),
        pltpu.VMEM_SHARED((DACC,), _F32),
        pltpu.VMEM_SHARED((DACC,), _F32),
        pltpu.SemaphoreType.DMA,
        pltpu.SemaphoreType.DMA,
        pltpu.SemaphoreType.DMA,
        pltpu.SemaphoreType.DMA,
        pltpu.SemaphoreType.DMA,
        pltpu.SemaphoreType.DMA,
    ],
)
def _sc1(*args):
    _sc1_body(*args)


def _tc2_body(nA_ref, nB_ref, den_ref, deg_ref, b1_ref, w2_ref,
              g_ref, dinv_ref):
    x1 = jnp.concatenate([nA_ref[...], nB_ref[...]], axis=1)
    x1 = jnp.maximum(x1 / (den_ref[...] + 1e-16) + b1_ref[...], 0.0)
    dg = deg_ref[...]
    dv = jnp.where(dg > 0, lax.rsqrt(jnp.maximum(dg, 1e-12)), 0.0)
    g_ref[...] = _mm(x1, w2_ref[...]) * dv
    dinv_ref[...] = dv


def _tc2(nA, nB, den, deg, b1r, W2):
    mb = 1000
    return pl.pallas_call(
        _tc2_body,
        grid=(N // mb,),
        in_specs=[
            pl.BlockSpec((mb, D_IN), lambda i: (i, 0)),
            pl.BlockSpec((mb, D_IN), lambda i: (i, 0)),
            pl.BlockSpec((mb, 1), lambda i: (i, 0)),
            pl.BlockSpec((mb, 1), lambda i: (i, 0)),
            pl.BlockSpec((1, H), lambda i: (0, 0)),
            pl.BlockSpec((H, D_OUT), lambda i: (0, 0)),
        ],
        out_specs=[
            pl.BlockSpec((mb, D_OUT), lambda i: (i, 0)),
            pl.BlockSpec((mb, 1), lambda i: (i, 0)),
        ],
        out_shape=[
            jax.ShapeDtypeStruct((N, D_OUT), _F32),
            jax.ShapeDtypeStruct((N, 1), _F32),
        ],
    )(nA, nB, den, deg, b1r, W2)


def _sc2_body(src_hbm, dst_hbm, g_hbm, accA_out, accB_out,
              sr0, sr1, dr0, dr1, dsc0, dsc1, rows0, rows1, acc,
              gsem0, gsem1, ssem0, ssem1, isem0, isem1):
    c = lax.axis_index("c")
    s = lax.axis_index("s")

    zero16 = jnp.zeros((16,), _F32)

    def _zrows(r, carry):
        for v in range(8):
            rows0[r, pl.ds(v * 16, 16)] = zero16
        return carry
    lax.fori_loop(0, BLK, _zrows, 0)

    r0 = pl.multiple_of(s * 632, 8)

    @pl.when(s < 15)
    def _():
        for k in range(4):
            pltpu.sync_copy(rows0, acc.at[pl.ds(r0 + k * 128, 128)])
        pltpu.sync_copy(rows0.at[pl.ds(0, 120)], acc.at[pl.ds(r0 + 512, 120)])

    @pl.when(s == 15)
    def _():
        for k in range(4):
            pltpu.sync_copy(rows0, acc.at[pl.ds(9480 + k * 128, 128)])
        pltpu.sync_copy(rows0.at[pl.ds(0, 24)], acc.at[pl.ds(9992, 24)])
    plsc.subcore_barrier()

    w0 = (c * NS + s) * EPS2

    def ibase(b):
        return pl.multiple_of(w0 + b * BLK, 128)

    def idx_start(b, sr, dr, isem):
        pltpu.async_copy(src_hbm.at[pl.ds(ibase(b), BLK)], sr, isem)
        pltpu.async_copy(dst_hbm.at[pl.ds(ibase(b), BLK)], dr.at[0], isem)

    def idx_wait(b, sr, dr, isem):
        pltpu.make_async_copy(src_hbm.at[pl.ds(ibase(b), BLK)], sr, isem).wait()
        pltpu.make_async_copy(dst_hbm.at[pl.ds(ibase(b), BLK)], dr.at[0],
                              isem).wait()

    def stage_dst(dr, dsc):
        for v in range(8):
            dsc[0, pl.ds(v * 16, 16)] = dr[0, pl.ds(v * 16, 16)]

    pltpu.sync_copy(src_hbm.at[pl.ds(ibase(0), BLK)], sr0)
    pltpu.sync_copy(dst_hbm.at[pl.ds(ibase(0), BLK)], dr0.at[0])
    stage_dst(dr0, dsc0)
    pltpu.async_copy(g_hbm.at[sr0], rows0, gsem0)
    idx_start(1, sr1, dr1, isem1)

    def _pair(qq, carry):
        b0 = qq * 2
        pltpu.make_async_copy(g_hbm.at[sr0], rows0, gsem0).wait()

        @pl.when(b0 > 0)
        def _():
            pltpu.make_async_copy(rows1, acc.at[dsc1.at[0]], ssem1).wait()
        idx_wait(b0 + 1, sr1, dr1, isem1)
        stage_dst(dr1, dsc1)
        pltpu.async_copy(g_hbm.at[sr1], rows1, gsem1)

        @pl.when(b0 + 2 < NBLK2)
        def _():
            idx_start(b0 + 2, sr0, dr0, isem0)
        pltpu.async_copy(rows0, acc.at[dsc0.at[0]], ssem0, add=True)

        pltpu.make_async_copy(g_hbm.at[sr1], rows1, gsem1).wait()
        pltpu.make_async_copy(rows0, acc.at[dsc0.at[0]], ssem0).wait()

        @pl.when(b0 + 2 < NBLK2)
        def _():
            idx_wait(b0 + 2, sr0, dr0, isem0)
            stage_dst(dr0, dsc0)
            pltpu.async_copy(g_hbm.at[sr0], rows0, gsem0)

        @pl.when(b0 + 3 < NBLK2)
        def _():
            idx_start(b0 + 3, sr1, dr1, isem1)
        pltpu.async_copy(rows1, acc.at[dsc1.at[0]], ssem1, add=True)
        return carry

    lax.fori_loop(0, NBLK2 // 2, _pair, 0)
    pltpu.make_async_copy(rows1, acc.at[dsc1.at[0]], ssem1).wait()
    plsc.subcore_barrier()

    q0 = pl.multiple_of(s * 624, 8)

    def _copy_out(dst_ref):
        @pl.when(s < 15)
        def _():
            pltpu.sync_copy(acc.at[pl.ds(q0, 624)], dst_ref.at[pl.ds(q0, 624)])

        @pl.when(s == 15)
        def _():
            pltpu.sync_copy(acc.at[pl.ds(9360, 640)], dst_ref.at[pl.ds(9360, 640)])

    @pl.when(c == 0)
    def _():
        _copy_out(accA_out)

    @pl.when(c == 1)
    def _():
        _copy_out(accB_out)


@functools.partial(
    pl.kernel,
    out_type=[
        jax.ShapeDtypeStruct((N, D_OUT), _F32),
        jax.ShapeDtypeStruct((N, D_OUT), _F32),
    ],
    mesh=plsc.VectorSubcoreMesh(core_axis_name="c", subcore_axis_name="s"),
    compiler_params=pltpu.CompilerParams(needs_layout_passes=False),
    scratch_types=[
        pltpu.VMEM((BLK,), _I32),
        pltpu.VMEM((BLK,), _I32),
        pltpu.VMEM((1, BLK), _I32),
        pltpu.VMEM((1, BLK), _I32),
        pltpu.VMEM((1, BLK), _I32),
        pltpu.VMEM((1, BLK), _I32),
        pltpu.VMEM((BLK, D_OUT), _F32),
        pltpu.VMEM((BLK, D_OUT), _F32),
        pltpu.VMEM_SHARED((NACC, D_OUT), _F32),
        pltpu.SemaphoreType.DMA,
        pltpu.SemaphoreType.DMA,
        pltpu.SemaphoreType.DMA,
        pltpu.SemaphoreType.DMA,
        pltpu.SemaphoreType.DMA,
        pltpu.SemaphoreType.DMA,
    ],
)
def _sc2(*args):
    _sc2_body(*args)


def _tc3_body(aA_ref, aB_ref, dinv_ref, b2_ref, z_ref):
    z_ref[...] = ((aA_ref[...] + aB_ref[...]) * dinv_ref[...] + b2_ref[...])


def _tc3(aA, aB, dinv, b2r):
    mb = 1000
    return pl.pallas_call(
        _tc3_body,
        grid=(N // mb,),
        in_specs=[
            pl.BlockSpec((mb, D_OUT), lambda i: (i, 0)),
            pl.BlockSpec((mb, D_OUT), lambda i: (i, 0)),
            pl.BlockSpec((mb, 1), lambda i: (i, 0)),
            pl.BlockSpec((1, D_OUT), lambda i: (0, 0)),
        ],
        out_specs=pl.BlockSpec((mb, D_OUT), lambda i: (i, 0)),
        out_shape=jax.ShapeDtypeStruct((N, D_OUT), _F32),
    )(aA, aB, dinv, b2r)


def kernel(x, edge_index, W1, a_src, a_dst, b1, W2, b2):
    loop = jnp.arange(N, dtype=_I32)
    src = jnp.concatenate([edge_index[0].astype(_I32), loop])
    dst = jnp.concatenate([edge_index[1].astype(_I32), loop])
    npad = E2PAD - E2
    src_pad = jnp.concatenate([src, jnp.zeros((npad,), _I32)])
    dst_pad = jnp.concatenate([dst, jnp.full((npad,), N, _I32)])

    A2 = jnp.stack([a_src, a_dst], axis=1)
    hcat, ea = _tc1(x, W1, A2)
    es_pad = jnp.zeros((NTAB,), _F32).at[:N].set(ea[:, 0])
    ed_pad = jnp.zeros((NTAB,), _F32).at[:N].set(ea[:, 1])

    numA, numB, den2, deg2 = _sc1(src_pad, dst_pad, es_pad, ed_pad, hcat)
    den = den2[:N].reshape(N, 1)
    deg = deg2[:N].reshape(N, 1)

    g, dinv = _tc2(numA, numB, den, deg, b1.reshape(1, H), W2)
    accA, accB = _sc2(src_pad, dst_pad, g)
    return _tc3(accA, accB, dinv, b2.reshape(1, D_OUT))


# speedup vs baseline: 17.6991x; 1.0567x over previous
import functools

import jax
import jax.numpy as jnp
from jax import lax
from jax.experimental import pallas as pl
from jax.experimental.pallas import tpu as pltpu
from jax.experimental.pallas import tpu_sc as plsc

N = 10000
E = 320000
E2 = E + N
D_IN = 128
H = 256
D_OUT = 128

NC = 2
NS = 16
BLK = 128
BLK1 = 64
EPS = 20736
NBLK = EPS // BLK1
EPS2 = 10496
NBLK2 = EPS2 // BLK
E2PAD = EPS2 * 32
NTAB = 10016
NACC = 10016
DACC = 10240

_F32 = jnp.float32
_I32 = jnp.int32


def _bcast_lane(vec16, j):
    idx = jnp.full((16, 1), j, _I32)
    return lax.gather(
        vec16, idx,
        lax.GatherDimensionNumbers(offset_dims=(), collapsed_slice_dims=(0,),
                                   start_index_map=(0,)),
        slice_sizes=(1,), mode=lax.GatherScatterMode.PROMISE_IN_BOUNDS)


def _mm(a, b):
    return lax.dot_general(a, b, (((1,), (0,)), ((), ())),
                           preferred_element_type=_F32)


def _tc1_body(x_ref, w1_ref, a2_ref, hcat_ref, ea_ref):
    j = pl.program_id(1)
    h = _mm(x_ref[...], w1_ref[...])
    hcat_ref[...] = h
    part = _mm(h, a2_ref[...])

    @pl.when(j == 0)
    def _():
        ea_ref[...] = part

    @pl.when(j != 0)
    def _():
        ea_ref[...] = ea_ref[...] + part


def _tc1(x, W1, A2):
    mb = 1000
    return pl.pallas_call(
        _tc1_body,
        grid=(N // mb, 2),
        in_specs=[
            pl.BlockSpec((mb, D_IN), lambda i, j: (i, 0)),
            pl.BlockSpec((D_IN, H // 2), lambda i, j: (0, j)),
            pl.BlockSpec((H // 2, 2), lambda i, j: (j, 0)),
        ],
        out_specs=[
            pl.BlockSpec((mb, D_IN), lambda i, j: (j * (N // mb) + i, 0)),
            pl.BlockSpec((mb, 2), lambda i, j: (i, 0)),
        ],
        out_shape=[
            jax.ShapeDtypeStruct((2 * N, D_IN), _F32),
            jax.ShapeDtypeStruct((N, 2), _F32),
        ],
        compiler_params=pltpu.CompilerParams(
            dimension_semantics=("parallel", "arbitrary")),
    )(x, W1, A2)


def _sc1_body(src_hbm, dst_hbm, es_hbm, ed_hbm, hcat_hbm,
              numerA, numerB, den_out, deg_out,
              es_t, ed_t,
              sr0, sr1, ssc0, ssc1, sa0, sa1,
              dr0, dr1, dsc0, dsc1, pv0, pv1, ones_v, zrow,
              rows0, rows1, acc, dacc, gacc,
              gsem0, gsem1, ssem0, ssem1, isem0, isem1):
    c = lax.axis_index("c")
    s = lax.axis_index("s")

    pltpu.sync_copy(es_hbm, es_t)
    pltpu.sync_copy(ed_hbm, ed_t)

    zero16 = jnp.zeros((16,), _F32)
    one16 = jnp.ones((16,), _F32)
    for v in range(8):
        zrow[0, pl.ds(v * 16, 16)] = zero16
    for v in range(BLK1 // 16):
        ones_v[pl.ds(v * 16, 16)] = one16

    def _zrows(r, carry):
        for v in range(8):
            rows0[r, pl.ds(v * 16, 16)] = zero16
        return carry
    lax.fori_loop(0, BLK1, _zrows, 0)

    r0 = pl.multiple_of(s * 632, 8)

    @pl.when(s < 15)
    def _():
        for k in range(9):
            pltpu.sync_copy(rows0, acc.at[pl.ds(r0 + k * 64, 64)])
        pltpu.sync_copy(rows0.at[pl.ds(0, 56)], acc.at[pl.ds(r0 + 576, 56)])

    @pl.when(s == 15)
    def _():
        for k in range(8):
            pltpu.sync_copy(rows0, acc.at[pl.ds(9480 + k * 64, 64)])
        pltpu.sync_copy(rows0.at[pl.ds(0, 24)], acc.at[pl.ds(9992, 24)])

    for k in range(5):
        off = pl.multiple_of(s * 640 + k * 128, 128)
        pltpu.sync_copy(zrow.at[0], dacc.at[pl.ds(off, 128)])
        pltpu.sync_copy(zrow.at[0], gacc.at[pl.ds(off, 128)])
    plsc.subcore_barrier()

    coff = c * N

    def ibase(b):
        return pl.multiple_of(s * EPS + b * BLK1, 64)

    def idx_start(b, sr, dr, isem):
        pltpu.async_copy(src_hbm.at[pl.ds(ibase(b), BLK1)], sr, isem)
        pltpu.async_copy(dst_hbm.at[pl.ds(ibase(b), BLK1)], dr.at[0], isem)

    def idx_wait(b, sr, dr, isem):
        pltpu.make_async_copy(src_hbm.at[pl.ds(ibase(b), BLK1)], sr, isem).wait()
        pltpu.make_async_copy(dst_hbm.at[pl.ds(ibase(b), BLK1)], dr.at[0],
                              isem).wait()

    def stage_idx(sr, dr, ssc, dsc, sa):
        for v in range(BLK1 // 16):
            sv = sr[pl.ds(v * 16, 16)]
            ssc[pl.ds(v * 16, 16)] = sv
            sa[pl.ds(v * 16, 16)] = sv + coff
            dsc[0, pl.ds(v * 16, 16)] = dr[0, pl.ds(v * 16, 16)]

    def gather_start(sa, rows, gsem):
        pltpu.async_copy(hcat_hbm.at[sa], rows, gsem)

    def gather_wait(sa, rows, gsem):
        pltpu.make_async_copy(hcat_hbm.at[sa], rows, gsem).wait()

    def scatter_start(rows, dsc, ssem):
        pltpu.async_copy(rows, acc.at[dsc.at[0]], ssem, add=True)

    def scatter_wait(rows, dsc, ssem):
        pltpu.make_async_copy(rows, acc.at[dsc.at[0]], ssem).wait()

    def compute_scale(ssc, dsc, pvr, rows):
        def inner(v, carry):
            sv = ssc[pl.ds(v * 16, 16)]
            dv = dsc[0, pl.ds(v * 16, 16)]
            t = plsc.load_gather(es_t, [sv]) + plsc.load_gather(ed_t, [dv])
            p16 = jnp.exp(jnp.maximum(t, 0.2 * t))
            pvr[pl.ds(v * 16, 16)] = p16
            for j in range(16):
                pvb = _bcast_lane(p16, j)
                r = v * 16 + j
                for q in range(8):
                    rows[r, pl.ds(q * 16, 16)] = (
                        rows[r, pl.ds(q * 16, 16)] * pvb)
            return carry
        lax.fori_loop(0, BLK1 // 16, inner, 0)

    def small_scatters(pvr, dsc):
        @pl.when(c == 0)
        def _():
            pltpu.sync_copy(pvr, dacc.at[dsc.at[0]], add=True)
            pltpu.sync_copy(ones_v, gacc.at[dsc.at[0]], add=True)

    pltpu.sync_copy(src_hbm.at[pl.ds(ibase(0), BLK1)], sr0)
    pltpu.sync_copy(dst_hbm.at[pl.ds(ibase(0), BLK1)], dr0.at[0])
    stage_idx(sr0, dr0, ssc0, dsc0, sa0)
    gather_start(sa0, rows0, gsem0)
    idx_start(1, sr1, dr1, isem1)

    def _pair(qq, carry):
        b0 = qq * 2
        gather_wait(sa0, rows0, gsem0)

        @pl.when(b0 > 0)
        def _():
            scatter_wait(rows1, dsc1, ssem1)
        idx_wait(b0 + 1, sr1, dr1, isem1)
        stage_idx(sr1, dr1, ssc1, dsc1, sa1)
        gather_start(sa1, rows1, gsem1)

        @pl.when(b0 + 2 < NBLK)
        def _():
            idx_start(b0 + 2, sr0, dr0, isem0)
        compute_scale(ssc0, dsc0, pv0, rows0)
        scatter_start(rows0, dsc0, ssem0)
        small_scatters(pv0, dsc0)

        gather_wait(sa1, rows1, gsem1)
        scatter_wait(rows0, dsc0, ssem0)

        @pl.when(b0 + 2 < NBLK)
        def _():
            idx_wait(b0 + 2, sr0, dr0, isem0)
            stage_idx(sr0, dr0, ssc0, dsc0, sa0)
            gather_start(sa0, rows0, gsem0)

        @pl.when(b0 + 3 < NBLK)
        def _():
            idx_start(b0 + 3, sr1, dr1, isem1)
        compute_scale(ssc1, dsc1, pv1, rows1)
        scatter_start(rows1, dsc1, ssem1)
        small_scatters(pv1, dsc1)
        return carry

    lax.fori_loop(0, NBLK // 2, _pair, 0)
    scatter_wait(rows1, dsc1, ssem1)
    plsc.subcore_barrier()

    q0 = pl.multiple_of(s * 624, 8)

    def _copy_num(dst_ref):
        @pl.when(s < 15)
        def _():
            pltpu.sync_copy(acc.at[pl.ds(q0, 624)], dst_ref.at[pl.ds(q0, 624)])

        @pl.when(s == 15)
        def _():
            pltpu.sync_copy(acc.at[pl.ds(9360, 640)], dst_ref.at[pl.ds(9360, 640)])

    @pl.when(c == 0)
    def _():
        _copy_num(numerA)
        doff = pl.multiple_of(s * 640, 128)
        pltpu.sync_copy(dacc.at[pl.ds(doff, 640)], den_out.at[pl.ds(doff, 640)])
        pltpu.sync_copy(gacc.at[pl.ds(doff, 640)], deg_out.at[pl.ds(doff, 640)])

    @pl.when(c == 1)
    def _():
        _copy_num(numerB)


@functools.partial(
    pl.kernel,
    out_type=[
        jax.ShapeDtypeStruct((N, D_IN), _F32),
        jax.ShapeDtypeStruct((N, D_IN), _F32),
        jax.ShapeDtypeStruct((DACC,), _F32),
        jax.ShapeDtypeStruct((DACC,), _F32),
    ],
    mesh=plsc.VectorSubcoreMesh(core_axis_name="c", subcore_axis_name="s"),
    compiler_params=pltpu.CompilerParams(needs_layout_passes=False),
    scratch_types=[
        pltpu.VMEM((NTAB,), _F32),
        pltpu.VMEM((NTAB,), _F32),
        pltpu.VMEM((BLK1,), _I32),
        pltpu.VMEM((BLK1,), _I32),
        pltpu.VMEM((BLK1,), _I32),
        pltpu.VMEM((BLK1,), _I32),
        pltpu.VMEM((BLK1,), _I32),
        pltpu.VMEM((BLK1,), _I32),
        pltpu.VMEM((1, BLK1), _I32),
        pltpu.VMEM((1, BLK1), _I32),
        pltpu.VMEM((1, BLK1), _I32),
        pltpu.VMEM((1, BLK1), _I32),
        pltpu.VMEM((BLK1,), _F32),
        pltpu.VMEM((BLK1,), _F32),
        pltpu.VMEM((BLK1,), _F32),
        pltpu.VMEM((1, 128), _F32),
        pltpu.VMEM((BLK1, D_IN), _F32),
        pltpu.VMEM((BLK1, D_IN), _F32),
        pltpu.VMEM_SHARED((NACC, D_IN), _F32),
        pltpu.VMEM_SHARED((DACC,), _F32),
        pltpu.VMEM_SHARED((DACC,), _F32),
        pltpu.SemaphoreType.DMA,
        pltpu.SemaphoreType.DMA,
        pltpu.SemaphoreType.DMA,
        pltpu.SemaphoreType.DMA,
        pltpu.SemaphoreType.DMA,
        pltpu.SemaphoreType.DMA,
    ],
)
def _sc1(*args):
    _sc1_body(*args)


def _tc2_body(nA_ref, nB_ref, den_ref, deg_ref, b1_ref, w2_ref,
              g_ref, dinv_ref):
    x1 = jnp.concatenate([nA_ref[...], nB_ref[...]], axis=1)
    x1 = jnp.maximum(x1 / (den_ref[...] + 1e-16) + b1_ref[...], 0.0)
    dg = deg_ref[...]
    dv = jnp.where(dg > 0, lax.rsqrt(jnp.maximum(dg, 1e-12)), 0.0)
    g_ref[...] = _mm(x1, w2_ref[...]) * dv
    dinv_ref[...] = dv


def _tc2(nA, nB, den, deg, b1r, W2):
    mb = 1000
    return pl.pallas_call(
        _tc2_body,
        grid=(N // mb,),
        in_specs=[
            pl.BlockSpec((mb, D_IN), lambda i: (i, 0)),
            pl.BlockSpec((mb, D_IN), lambda i: (i, 0)),
            pl.BlockSpec((mb, 1), lambda i: (i, 0)),
            pl.BlockSpec((mb, 1), lambda i: (i, 0)),
            pl.BlockSpec((1, H), lambda i: (0, 0)),
            pl.BlockSpec((H, D_OUT), lambda i: (0, 0)),
        ],
        out_specs=[
            pl.BlockSpec((mb, D_OUT), lambda i: (i, 0)),
            pl.BlockSpec((mb, 1), lambda i: (i, 0)),
        ],
        out_shape=[
            jax.ShapeDtypeStruct((N, D_OUT), _F32),
            jax.ShapeDtypeStruct((N, 1), _F32),
        ],
    )(nA, nB, den, deg, b1r, W2)


def _sc2_body(src_hbm, dst_hbm, g_hbm, accA_out, accB_out,
              sr0, sr1, dr0, dr1, dsc0, dsc1, rows0, rows1, acc,
              gsem0, gsem1, ssem0, ssem1, isem0, isem1):
    c = lax.axis_index("c")
    s = lax.axis_index("s")

    zero16 = jnp.zeros((16,), _F32)

    def _zrows(r, carry):
        for v in range(8):
            rows0[r, pl.ds(v * 16, 16)] = zero16
        return carry
    lax.fori_loop(0, BLK, _zrows, 0)

    r0 = pl.multiple_of(s * 632, 8)

    @pl.when(s < 15)
    def _():
        for k in range(4):
            pltpu.sync_copy(rows0, acc.at[pl.ds(r0 + k * 128, 128)])
        pltpu.sync_copy(rows0.at[pl.ds(0, 120)], acc.at[pl.ds(r0 + 512, 120)])

    @pl.when(s == 15)
    def _():
        for k in range(4):
            pltpu.sync_copy(rows0, acc.at[pl.ds(9480 + k * 128, 128)])
        pltpu.sync_copy(rows0.at[pl.ds(0, 24)], acc.at[pl.ds(9992, 24)])
    plsc.subcore_barrier()

    w0 = (c * NS + s) * EPS2

    def ibase(b):
        return pl.multiple_of(w0 + b * BLK, 128)

    def idx_start(b, sr, dr, isem):
        pltpu.async_copy(src_hbm.at[pl.ds(ibase(b), BLK)], sr, isem)
        pltpu.async_copy(dst_hbm.at[pl.ds(ibase(b), BLK)], dr.at[0], isem)

    def idx_wait(b, sr, dr, isem):
        pltpu.make_async_copy(src_hbm.at[pl.ds(ibase(b), BLK)], sr, isem).wait()
        pltpu.make_async_copy(dst_hbm.at[pl.ds(ibase(b), BLK)], dr.at[0],
                              isem).wait()

    def stage_dst(dr, dsc):
        for v in range(8):
            dsc[0, pl.ds(v * 16, 16)] = dr[0, pl.ds(v * 16, 16)]

    pltpu.sync_copy(src_hbm.at[pl.ds(ibase(0), BLK)], sr0)
    pltpu.sync_copy(dst_hbm.at[pl.ds(ibase(0), BLK)], dr0.at[0])
    stage_dst(dr0, dsc0)
    pltpu.async_copy(g_hbm.at[sr0], rows0, gsem0)
    idx_start(1, sr1, dr1, isem1)

    def _pair(qq, carry):
        b0 = qq * 2
        pltpu.make_async_copy(g_hbm.at[sr0], rows0, gsem0).wait()

        @pl.when(b0 > 0)
        def _():
            pltpu.make_async_copy(rows1, acc.at[dsc1.at[0]], ssem1).wait()
        idx_wait(b0 + 1, sr1, dr1, isem1)
        stage_dst(dr1, dsc1)
        pltpu.async_copy(g_hbm.at[sr1], rows1, gsem1)

        @pl.when(b0 + 2 < NBLK2)
        def _():
            idx_start(b0 + 2, sr0, dr0, isem0)
        pltpu.async_copy(rows0, acc.at[dsc0.at[0]], ssem0, add=True)

        pltpu.make_async_copy(g_hbm.at[sr1], rows1, gsem1).wait()
        pltpu.make_async_copy(rows0, acc.at[dsc0.at[0]], ssem0).wait()

        @pl.when(b0 + 2 < NBLK2)
        def _():
            idx_wait(b0 + 2, sr0, dr0, isem0)
            stage_dst(dr0, dsc0)
            pltpu.async_copy(g_hbm.at[sr0], rows0, gsem0)

        @pl.when(b0 + 3 < NBLK2)
        def _():
            idx_start(b0 + 3, sr1, dr1, isem1)
        pltpu.async_copy(rows1, acc.at[dsc1.at[0]], ssem1, add=True)
        return carry

    lax.fori_loop(0, NBLK2 // 2, _pair, 0)
    pltpu.make_async_copy(rows1, acc.at[dsc1.at[0]], ssem1).wait()
    plsc.subcore_barrier()

    q0 = pl.multiple_of(s * 624, 8)

    def _copy_out(dst_ref):
        @pl.when(s < 15)
        def _():
            pltpu.sync_copy(acc.at[pl.ds(q0, 624)], dst_ref.at[pl.ds(q0, 624)])

        @pl.when(s == 15)
        def _():
            pltpu.sync_copy(acc.at[pl.ds(9360, 640)], dst_ref.at[pl.ds(9360, 640)])

    @pl.when(c == 0)
    def _():
        _copy_out(accA_out)

    @pl.when(c == 1)
    def _():
        _copy_out(accB_out)


@functools.partial(
    pl.kernel,
    out_type=[
        jax.ShapeDtypeStruct((N, D_OUT), _F32),
        jax.ShapeDtypeStruct((N, D_OUT), _F32),
    ],
    mesh=plsc.VectorSubcoreMesh(core_axis_name="c", subcore_axis_name="s"),
    compiler_params=pltpu.CompilerParams(needs_layout_passes=False),
    scratch_types=[
        pltpu.VMEM((BLK,), _I32),
        pltpu.VMEM((BLK,), _I32),
        pltpu.VMEM((1, BLK), _I32),
        pltpu.VMEM((1, BLK), _I32),
        pltpu.VMEM((1, BLK), _I32),
        pltpu.VMEM((1, BLK), _I32),
        pltpu.VMEM((BLK, D_OUT), _F32),
        pltpu.VMEM((BLK, D_OUT), _F32),
        pltpu.VMEM_SHARED((NACC, D_OUT), _F32),
        pltpu.SemaphoreType.DMA,
        pltpu.SemaphoreType.DMA,
        pltpu.SemaphoreType.DMA,
        pltpu.SemaphoreType.DMA,
        pltpu.SemaphoreType.DMA,
        pltpu.SemaphoreType.DMA,
    ],
)
def _sc2(*args):
    _sc2_body(*args)


def _tc3_body(aA_ref, aB_ref, dinv_ref, b2_ref, z_ref):
    z_ref[...] = ((aA_ref[...] + aB_ref[...]) * dinv_ref[...] + b2_ref[...])


def _tc3(aA, aB, dinv, b2r):
    mb = 1000
    return pl.pallas_call(
        _tc3_body,
        grid=(N // mb,),
        in_specs=[
            pl.BlockSpec((mb, D_OUT), lambda i: (i, 0)),
            pl.BlockSpec((mb, D_OUT), lambda i: (i, 0)),
            pl.BlockSpec((mb, 1), lambda i: (i, 0)),
            pl.BlockSpec((1, D_OUT), lambda i: (0, 0)),
        ],
        out_specs=pl.BlockSpec((mb, D_OUT), lambda i: (i, 0)),
        out_shape=jax.ShapeDtypeStruct((N, D_OUT), _F32),
    )(aA, aB, dinv, b2r)


def kernel(x, edge_index, W1, a_src, a_dst, b1, W2, b2):
    loop = jnp.arange(N, dtype=_I32)
    src = jnp.concatenate([edge_index[0].astype(_I32), loop])
    dst = jnp.concatenate([edge_index[1].astype(_I32), loop])
    npad = E2PAD - E2
    src_pad = jnp.concatenate([src, jnp.zeros((npad,), _I32)])
    junk = N + (jnp.arange(npad, dtype=_I32) % 16)
    dst_pad = jnp.concatenate([dst, junk])

    A2 = jnp.stack([a_src, a_dst], axis=1)
    hcat, ea = _tc1(x, W1, A2)
    es_pad = jnp.zeros((NTAB,), _F32).at[:N].set(ea[:, 0])
    ed_pad = jnp.zeros((NTAB,), _F32).at[:N].set(ea[:, 1])

    numA, numB, den2, deg2 = _sc1(src_pad, dst_pad, es_pad, ed_pad, hcat)
    den = den2[:N].reshape(N, 1)
    deg = deg2[:N].reshape(N, 1)

    g, dinv = _tc2(numA, numB, den, deg, b1.reshape(1, H), W2)
    accA, accB = _sc2(src_pad, dst_pad, g)
    return _tc3(accA, accB, dinv, b2.reshape(1, D_OUT))


# speedup vs baseline: 28.4205x; 1.6058x over previous
import functools

import jax
import jax.numpy as jnp
from jax import lax
from jax.experimental import pallas as pl
from jax.experimental.pallas import tpu as pltpu
from jax.experimental.pallas import tpu_sc as plsc

N = 10000
E = 320000
E2 = E + N
D_IN = 128
H = 256
D_OUT = 128

NC = 2
NS = 16
BLK = 128
BLK1 = 64
EPS = 20736
NBLK = EPS // BLK1
EPS2 = 10496
NBLK2 = EPS2 // BLK
E2PAD = EPS2 * 32
NTAB = 10016
NACC = 10016
DACC = 10240

_F32 = jnp.float32
_I32 = jnp.int32


def _bcast_lane(vec16, j):
    idx = jnp.full((16, 1), j, _I32)
    return lax.gather(
        vec16, idx,
        lax.GatherDimensionNumbers(offset_dims=(), collapsed_slice_dims=(0,),
                                   start_index_map=(0,)),
        slice_sizes=(1,), mode=lax.GatherScatterMode.PROMISE_IN_BOUNDS)


def _mm(a, b):
    return lax.dot_general(a, b, (((1,), (0,)), ((), ())),
                           preferred_element_type=_F32)


def _tc1_body(x_ref, w1_ref, a2_ref, hcat_ref, ea_ref):
    j = pl.program_id(1)
    h = _mm(x_ref[...], w1_ref[...])
    hcat_ref[...] = h
    part = _mm(h, a2_ref[...])

    @pl.when(j == 0)
    def _():
        ea_ref[...] = part

    @pl.when(j != 0)
    def _():
        ea_ref[...] = ea_ref[...] + part


def _tc1(x, W1, A2):
    mb = 1000
    return pl.pallas_call(
        _tc1_body,
        grid=(N // mb, 2),
        in_specs=[
            pl.BlockSpec((mb, D_IN), lambda i, j: (i, 0)),
            pl.BlockSpec((D_IN, H // 2), lambda i, j: (0, j)),
            pl.BlockSpec((H // 2, 2), lambda i, j: (j, 0)),
        ],
        out_specs=[
            pl.BlockSpec((mb, D_IN), lambda i, j: (j * (N // mb) + i, 0)),
            pl.BlockSpec((mb, 2), lambda i, j: (i, 0)),
        ],
        out_shape=[
            jax.ShapeDtypeStruct((2 * N, D_IN), _F32),
            jax.ShapeDtypeStruct((N, 2), _F32),
        ],
        compiler_params=pltpu.CompilerParams(
            dimension_semantics=("parallel", "arbitrary")),
    )(x, W1, A2)


def _sc1_body(src_hbm, dst_hbm, es_hbm, ed_hbm, hcat_hbm,
              numerA, numerB, den_out, deg_out,
              es_t, ed_t,
              sr0, sr1, ssc0, ssc1, sa0, sa1,
              dr0, dr1, dsc0, dsc1, pv0, pv1, ones_v, zrow,
              rows0, rows1, acc, dacc, gacc,
              gsem0, gsem1, ssem0, ssem1, isem0, isem1):
    c = lax.axis_index("c")
    s = lax.axis_index("s")

    pltpu.sync_copy(es_hbm, es_t)
    pltpu.sync_copy(ed_hbm, ed_t)

    zero16 = jnp.zeros((16,), _F32)
    one16 = jnp.ones((16,), _F32)
    for v in range(8):
        zrow[0, pl.ds(v * 16, 16)] = zero16
    for v in range(BLK1 // 16):
        ones_v[pl.ds(v * 16, 16)] = one16

    def _zrows(r, carry):
        for v in range(8):
            rows0[r, pl.ds(v * 16, 16)] = zero16
        return carry
    lax.fori_loop(0, BLK1, _zrows, 0)

    r0 = pl.multiple_of(s * 632, 8)

    @pl.when(s < 15)
    def _():
        for k in range(9):
            pltpu.sync_copy(rows0, acc.at[pl.ds(r0 + k * 64, 64)])
        pltpu.sync_copy(rows0.at[pl.ds(0, 56)], acc.at[pl.ds(r0 + 576, 56)])

    @pl.when(s == 15)
    def _():
        for k in range(8):
            pltpu.sync_copy(rows0, acc.at[pl.ds(9480 + k * 64, 64)])
        pltpu.sync_copy(rows0.at[pl.ds(0, 24)], acc.at[pl.ds(9992, 24)])

    for k in range(5):
        off = pl.multiple_of(s * 640 + k * 128, 128)
        pltpu.sync_copy(zrow.at[0], dacc.at[pl.ds(off, 128)])
        pltpu.sync_copy(zrow.at[0], gacc.at[pl.ds(off, 128)])
    plsc.subcore_barrier()

    coff = c * N

    def ibase(b):
        return pl.multiple_of(s * EPS + b * BLK1, 64)

    def idx_start(b, sr, dr, isem):
        pltpu.async_copy(src_hbm.at[pl.ds(ibase(b), BLK1)], sr, isem)
        pltpu.async_copy(dst_hbm.at[pl.ds(ibase(b), BLK1)], dr.at[0], isem)

    def idx_wait(b, sr, dr, isem):
        pltpu.make_async_copy(src_hbm.at[pl.ds(ibase(b), BLK1)], sr, isem).wait()
        pltpu.make_async_copy(dst_hbm.at[pl.ds(ibase(b), BLK1)], dr.at[0],
                              isem).wait()

    def stage_idx(sr, dr, ssc, dsc, sa):
        for v in range(BLK1 // 16):
            sv = sr[pl.ds(v * 16, 16)]
            ssc[pl.ds(v * 16, 16)] = sv
            sa[pl.ds(v * 16, 16)] = sv + coff
            dsc[0, pl.ds(v * 16, 16)] = dr[0, pl.ds(v * 16, 16)]

    def gather_start(sa, rows, gsem):
        pltpu.async_copy(hcat_hbm.at[sa], rows, gsem)

    def gather_wait(sa, rows, gsem):
        pltpu.make_async_copy(hcat_hbm.at[sa], rows, gsem).wait()

    def scatter_start(rows, dsc, ssem):
        pltpu.async_copy(rows, acc.at[dsc.at[0]], ssem, add=True)

    def scatter_wait(rows, dsc, ssem):
        pltpu.make_async_copy(rows, acc.at[dsc.at[0]], ssem).wait()

    def compute_scale(ssc, dsc, pvr, rows):
        def inner(v, carry):
            sv = ssc[pl.ds(v * 16, 16)]
            dv = dsc[0, pl.ds(v * 16, 16)]
            t = plsc.load_gather(es_t, [sv]) + plsc.load_gather(ed_t, [dv])
            p16 = jnp.exp(jnp.maximum(t, 0.2 * t))
            pvr[pl.ds(v * 16, 16)] = p16
            for j in range(16):
                pvb = _bcast_lane(p16, j)
                r = v * 16 + j
                for q in range(8):
                    rows[r, pl.ds(q * 16, 16)] = (
                        rows[r, pl.ds(q * 16, 16)] * pvb)
            return carry
        lax.fori_loop(0, BLK1 // 16, inner, 0)

    def small_scatters(pvr, dsc):
        @pl.when(c == 0)
        def _():
            pltpu.sync_copy(pvr, dacc.at[dsc.at[0]], add=True)
            pltpu.sync_copy(ones_v, gacc.at[dsc.at[0]], add=True)

    pltpu.sync_copy(src_hbm.at[pl.ds(ibase(0), BLK1)], sr0)
    pltpu.sync_copy(dst_hbm.at[pl.ds(ibase(0), BLK1)], dr0.at[0])
    stage_idx(sr0, dr0, ssc0, dsc0, sa0)
    gather_start(sa0, rows0, gsem0)
    idx_start(1, sr1, dr1, isem1)

    def _pair(qq, carry):
        b0 = qq * 2
        gather_wait(sa0, rows0, gsem0)

        @pl.when(b0 > 0)
        def _():
            scatter_wait(rows1, dsc1, ssem1)
        idx_wait(b0 + 1, sr1, dr1, isem1)
        stage_idx(sr1, dr1, ssc1, dsc1, sa1)
        gather_start(sa1, rows1, gsem1)

        @pl.when(b0 + 2 < NBLK)
        def _():
            idx_start(b0 + 2, sr0, dr0, isem0)
        compute_scale(ssc0, dsc0, pv0, rows0)
        scatter_start(rows0, dsc0, ssem0)
        small_scatters(pv0, dsc0)

        gather_wait(sa1, rows1, gsem1)
        scatter_wait(rows0, dsc0, ssem0)

        @pl.when(b0 + 2 < NBLK)
        def _():
            idx_wait(b0 + 2, sr0, dr0, isem0)
            stage_idx(sr0, dr0, ssc0, dsc0, sa0)
            gather_start(sa0, rows0, gsem0)

        @pl.when(b0 + 3 < NBLK)
        def _():
            idx_start(b0 + 3, sr1, dr1, isem1)
        compute_scale(ssc1, dsc1, pv1, rows1)
        scatter_start(rows1, dsc1, ssem1)
        small_scatters(pv1, dsc1)
        return carry

    lax.fori_loop(0, NBLK // 2, _pair, 0)
    scatter_wait(rows1, dsc1, ssem1)
    plsc.subcore_barrier()

    q0 = pl.multiple_of(s * 624, 8)

    def _copy_num(dst_ref):
        @pl.when(s < 15)
        def _():
            pltpu.sync_copy(acc.at[pl.ds(q0, 624)], dst_ref.at[pl.ds(q0, 624)])

        @pl.when(s == 15)
        def _():
            pltpu.sync_copy(acc.at[pl.ds(9360, 640)], dst_ref.at[pl.ds(9360, 640)])

    @pl.when(c == 0)
    def _():
        _copy_num(numerA)
        doff = pl.multiple_of(s * 640, 128)
        pltpu.sync_copy(dacc.at[pl.ds(doff, 640)], den_out.at[pl.ds(doff, 640)])
        pltpu.sync_copy(gacc.at[pl.ds(doff, 640)], deg_out.at[pl.ds(doff, 640)])

    @pl.when(c == 1)
    def _():
        _copy_num(numerB)


@functools.partial(
    pl.kernel,
    out_type=[
        jax.ShapeDtypeStruct((N, D_IN), _F32),
        jax.ShapeDtypeStruct((N, D_IN), _F32),
        jax.ShapeDtypeStruct((DACC,), _F32),
        jax.ShapeDtypeStruct((DACC,), _F32),
    ],
    mesh=plsc.VectorSubcoreMesh(core_axis_name="c", subcore_axis_name="s"),
    compiler_params=pltpu.CompilerParams(needs_layout_passes=False),
    scratch_types=[
        pltpu.VMEM((NTAB,), _F32),
        pltpu.VMEM((NTAB,), _F32),
        pltpu.VMEM((BLK1,), _I32),
        pltpu.VMEM((BLK1,), _I32),
        pltpu.VMEM((BLK1,), _I32),
        pltpu.VMEM((BLK1,), _I32),
        pltpu.VMEM((BLK1,), _I32),
        pltpu.VMEM((BLK1,), _I32),
        pltpu.VMEM((1, BLK1), _I32),
        pltpu.VMEM((1, BLK1), _I32),
        pltpu.VMEM((1, BLK1), _I32),
        pltpu.VMEM((1, BLK1), _I32),
        pltpu.VMEM((BLK1,), _F32),
        pltpu.VMEM((BLK1,), _F32),
        pltpu.VMEM((BLK1,), _F32),
        pltpu.VMEM((1, 128), _F32),
        pltpu.VMEM((BLK1, D_IN), _F32),
        pltpu.VMEM((BLK1, D_IN), _F32),
        pltpu.VMEM_SHARED((NACC, D_IN), _F32),
        pltpu.VMEM_SHARED((DACC,), _F32),
        pltpu.VMEM_SHARED((DACC,), _F32),
        pltpu.SemaphoreType.DMA,
        pltpu.SemaphoreType.DMA,
        pltpu.SemaphoreType.DMA,
        pltpu.SemaphoreType.DMA,
        pltpu.SemaphoreType.DMA,
        pltpu.SemaphoreType.DMA,
    ],
)
def _sc1(*args):
    _sc1_body(*args)


def _tc2_body(nA_ref, nB_ref, den_ref, deg_ref, b1_ref, w2_ref,
              g_ref, dinv_ref):
    x1 = jnp.concatenate([nA_ref[...], nB_ref[...]], axis=1)
    x1 = jnp.maximum(x1 / (den_ref[...] + 1e-16) + b1_ref[...], 0.0)
    dg = deg_ref[...]
    dv = jnp.where(dg > 0, lax.rsqrt(jnp.maximum(dg, 1e-12)), 0.0)
    g_ref[...] = _mm(x1, w2_ref[...]) * dv
    dinv_ref[...] = dv


def _tc2(nA, nB, den, deg, b1r, W2):
    mb = 1000
    return pl.pallas_call(
        _tc2_body,
        grid=(N // mb,),
        in_specs=[
            pl.BlockSpec((mb, D_IN), lambda i: (i, 0)),
            pl.BlockSpec((mb, D_IN), lambda i: (i, 0)),
            pl.BlockSpec((mb, 1), lambda i: (i, 0)),
            pl.BlockSpec((mb, 1), lambda i: (i, 0)),
            pl.BlockSpec((1, H), lambda i: (0, 0)),
            pl.BlockSpec((H, D_OUT), lambda i: (0, 0)),
        ],
        out_specs=[
            pl.BlockSpec((mb, D_OUT), lambda i: (i, 0)),
            pl.BlockSpec((mb, 1), lambda i: (i, 0)),
        ],
        out_shape=[
            jax.ShapeDtypeStruct((N, D_OUT), _F32),
            jax.ShapeDtypeStruct((N, 1), _F32),
        ],
    )(nA, nB, den, deg, b1r, W2)


def _sc2_body(src_hbm, dst_hbm, g_hbm, accA_out, accB_out,
              sr0, sr1, dr0, dr1, dsc0, dsc1, rows0, rows1, acc,
              gsem0, gsem1, ssem0, ssem1, isem0, isem1):
    c = lax.axis_index("c")
    s = lax.axis_index("s")

    zero16 = jnp.zeros((16,), _F32)

    def _zrows(r, carry):
        for v in range(8):
            rows0[r, pl.ds(v * 16, 16)] = zero16
        return carry
    lax.fori_loop(0, BLK, _zrows, 0)

    r0 = pl.multiple_of(s * 632, 8)

    @pl.when(s < 15)
    def _():
        for k in range(4):
            pltpu.sync_copy(rows0, acc.at[pl.ds(r0 + k * 128, 128)])
        pltpu.sync_copy(rows0.at[pl.ds(0, 120)], acc.at[pl.ds(r0 + 512, 120)])

    @pl.when(s == 15)
    def _():
        for k in range(4):
            pltpu.sync_copy(rows0, acc.at[pl.ds(9480 + k * 128, 128)])
        pltpu.sync_copy(rows0.at[pl.ds(0, 24)], acc.at[pl.ds(9992, 24)])
    plsc.subcore_barrier()

    w0 = (c * NS + s) * EPS2

    def ibase(b):
        return pl.multiple_of(w0 + b * BLK, 128)

    def idx_start(b, sr, dr, isem):
        pltpu.async_copy(src_hbm.at[pl.ds(ibase(b), BLK)], sr, isem)
        pltpu.async_copy(dst_hbm.at[pl.ds(ibase(b), BLK)], dr.at[0], isem)

    def idx_wait(b, sr, dr, isem):
        pltpu.make_async_copy(src_hbm.at[pl.ds(ibase(b), BLK)], sr, isem).wait()
        pltpu.make_async_copy(dst_hbm.at[pl.ds(ibase(b), BLK)], dr.at[0],
                              isem).wait()

    def stage_dst(dr, dsc):
        for v in range(8):
            dsc[0, pl.ds(v * 16, 16)] = dr[0, pl.ds(v * 16, 16)]

    pltpu.sync_copy(src_hbm.at[pl.ds(ibase(0), BLK)], sr0)
    pltpu.sync_copy(dst_hbm.at[pl.ds(ibase(0), BLK)], dr0.at[0])
    stage_dst(dr0, dsc0)
    pltpu.async_copy(g_hbm.at[sr0], rows0, gsem0)
    idx_start(1, sr1, dr1, isem1)

    def _pair(qq, carry):
        b0 = qq * 2
        pltpu.make_async_copy(g_hbm.at[sr0], rows0, gsem0).wait()

        @pl.when(b0 > 0)
        def _():
            pltpu.make_async_copy(rows1, acc.at[dsc1.at[0]], ssem1).wait()
        idx_wait(b0 + 1, sr1, dr1, isem1)
        stage_dst(dr1, dsc1)
        pltpu.async_copy(g_hbm.at[sr1], rows1, gsem1)

        @pl.when(b0 + 2 < NBLK2)
        def _():
            idx_start(b0 + 2, sr0, dr0, isem0)
        pltpu.async_copy(rows0, acc.at[dsc0.at[0]], ssem0, add=True)

        pltpu.make_async_copy(g_hbm.at[sr1], rows1, gsem1).wait()
        pltpu.make_async_copy(rows0, acc.at[dsc0.at[0]], ssem0).wait()

        @pl.when(b0 + 2 < NBLK2)
        def _():
            idx_wait(b0 + 2, sr0, dr0, isem0)
            stage_dst(dr0, dsc0)
            pltpu.async_copy(g_hbm.at[sr0], rows0, gsem0)

        @pl.when(b0 + 3 < NBLK2)
        def _():
            idx_start(b0 + 3, sr1, dr1, isem1)
        pltpu.async_copy(rows1, acc.at[dsc1.at[0]], ssem1, add=True)
        return carry

    lax.fori_loop(0, NBLK2 // 2, _pair, 0)
    pltpu.make_async_copy(rows1, acc.at[dsc1.at[0]], ssem1).wait()
    plsc.subcore_barrier()

    q0 = pl.multiple_of(s * 624, 8)

    def _copy_out(dst_ref):
        @pl.when(s < 15)
        def _():
            pltpu.sync_copy(acc.at[pl.ds(q0, 624)], dst_ref.at[pl.ds(q0, 624)])

        @pl.when(s == 15)
        def _():
            pltpu.sync_copy(acc.at[pl.ds(9360, 640)], dst_ref.at[pl.ds(9360, 640)])

    @pl.when(c == 0)
    def _():
        _copy_out(accA_out)

    @pl.when(c == 1)
    def _():
        _copy_out(accB_out)


@functools.partial(
    pl.kernel,
    out_type=[
        jax.ShapeDtypeStruct((N, D_OUT), _F32),
        jax.ShapeDtypeStruct((N, D_OUT), _F32),
    ],
    mesh=plsc.VectorSubcoreMesh(core_axis_name="c", subcore_axis_name="s"),
    compiler_params=pltpu.CompilerParams(needs_layout_passes=False),
    scratch_types=[
        pltpu.VMEM((BLK,), _I32),
        pltpu.VMEM((BLK,), _I32),
        pltpu.VMEM((1, BLK), _I32),
        pltpu.VMEM((1, BLK), _I32),
        pltpu.VMEM((1, BLK), _I32),
        pltpu.VMEM((1, BLK), _I32),
        pltpu.VMEM((BLK, D_OUT), _F32),
        pltpu.VMEM((BLK, D_OUT), _F32),
        pltpu.VMEM_SHARED((NACC, D_OUT), _F32),
        pltpu.SemaphoreType.DMA,
        pltpu.SemaphoreType.DMA,
        pltpu.SemaphoreType.DMA,
        pltpu.SemaphoreType.DMA,
        pltpu.SemaphoreType.DMA,
        pltpu.SemaphoreType.DMA,
    ],
)
def _sc2(*args):
    _sc2_body(*args)


def _tc3_body(aA_ref, aB_ref, dinv_ref, b2_ref, z_ref):
    z_ref[...] = ((aA_ref[...] + aB_ref[...]) * dinv_ref[...] + b2_ref[...])


def _tc3(aA, aB, dinv, b2r):
    mb = 1000
    return pl.pallas_call(
        _tc3_body,
        grid=(N // mb,),
        in_specs=[
            pl.BlockSpec((mb, D_OUT), lambda i: (i, 0)),
            pl.BlockSpec((mb, D_OUT), lambda i: (i, 0)),
            pl.BlockSpec((mb, 1), lambda i: (i, 0)),
            pl.BlockSpec((1, D_OUT), lambda i: (0, 0)),
        ],
        out_specs=pl.BlockSpec((mb, D_OUT), lambda i: (i, 0)),
        out_shape=jax.ShapeDtypeStruct((N, D_OUT), _F32),
    )(aA, aB, dinv, b2r)


def kernel(x, edge_index, W1, a_src, a_dst, b1, W2, b2):
    loop = jnp.arange(N, dtype=_I32)
    src = jnp.concatenate([edge_index[0].astype(_I32), loop])
    dst = jnp.concatenate([edge_index[1].astype(_I32), loop])
    npad = E2PAD - E2
    src_pad = jnp.concatenate([src, jnp.arange(npad, dtype=_I32)])
    junk = N + (jnp.arange(npad, dtype=_I32) % 16)
    dst_pad = jnp.concatenate([dst, junk])

    A2 = jnp.stack([a_src, a_dst], axis=1)
    hcat, ea = _tc1(x, W1, A2)
    es_pad = jnp.zeros((NTAB,), _F32).at[:N].set(ea[:, 0])
    ed_pad = jnp.zeros((NTAB,), _F32).at[:N].set(ea[:, 1])

    numA, numB, den2, deg2 = _sc1(src_pad, dst_pad, es_pad, ed_pad, hcat)
    den = den2[:N].reshape(N, 1)
    deg = deg2[:N].reshape(N, 1)

    g, dinv = _tc2(numA, numB, den, deg, b1.reshape(1, H), W2)
    accA, accB = _sc2(src_pad, dst_pad, g)
    return _tc3(accA, accB, dinv, b2.reshape(1, D_OUT))
